# baseline (device time: 248363 ns/iter reference)
import contextlib
import os

import jax
import jax.numpy as jnp
from jax import lax
from jax.experimental import pallas as pl
from jax.experimental.pallas import tpu as pltpu

_PROFILE = os.environ.get("KERNEL_PROFILE", "0") == "1"
_NO_COMM = os.environ.get("KERNEL_NO_COMM", "0") == "1"
_NO_COMPUTE = os.environ.get("KERNEL_NO_COMPUTE", "0") == "1"
_NO_LOAD = os.environ.get("KERNEL_NO_LOAD", "0") == "1"


def _scope(name):
    return jax.named_scope(name) if _PROFILE else contextlib.nullcontext()


N_DEV = 4
XC = 4
WC = 2


def kernel(x, w_mat, scale_x, scale_w):
    m_per, k = x.shape
    _, n = w_mat.shape
    n_per = n // N_DEV
    m_tot = m_per * N_DEV
    mx = m_per // XC
    kw = k // WC

    def body(x_hbm, w_hbm, sx_ref, sw_ref, out_ref,
             xstage, wstage, x8, w8, send_buf, recv_buf,
             xsems, wsems, send_sems, recv_sems):
        my = lax.axis_index("i")

        with _scope("barrier"):
            barrier_sem = pltpu.get_barrier_semaphore()
            for h in range(1, N_DEV):
                pl.semaphore_signal(
                    barrier_sem, inc=1,
                    device_id=((my + h) % N_DEV,),
                    device_id_type=pl.DeviceIdType.MESH,
                )
            pl.semaphore_wait(barrier_sem, N_DEV - 1)

        scale = sx_ref[0] * sw_ref[0]

        def start_x(c):
            cp = pltpu.make_async_copy(
                x_hbm.at[pl.ds(c * mx, mx), :],
                xstage.at[c % 2],
                xsems.at[c % 2],
            )
            cp.start()
            return cp

        def start_w(i):
            b, c = divmod(i, WC)
            j = (my + 1 + b) % N_DEV
            cp = pltpu.make_async_copy(
                w_hbm.at[pl.ds(c * kw, kw), pl.ds(j * n_per, n_per)],
                wstage.at[i % 2],
                wsems.at[i % 2],
            )
            cp.start()
            return cp

        if not _NO_LOAD:
            xcopies = {0: start_x(0), 1: start_x(1)}
            wcopies = {0: start_w(0), 1: start_w(1)}

            with _scope("x_drain"):
                for c in range(XC):
                    xcopies[c].wait()
                    x8[pl.ds(c * mx, mx), :] = xstage[c % 2, :, :].astype(
                        jnp.float8_e4m3fn)
                    if c + 2 < XC:
                        xcopies[c + 2] = start_x(c + 2)

        sends = []
        for b in range(N_DEV):
            j = (my + 1 + b) % N_DEV
            if not _NO_LOAD:
                with _scope(f"w_drain#b={b}"):
                    for c in range(WC):
                        i = b * WC + c
                        wcopies[i].wait()
                        w8[b, pl.ds(c * kw, kw), :] = wstage[
                            i % 2, :, :].astype(jnp.float8_e5m2)
                        if i + 2 < N_DEV * WC:
                            wcopies[i + 2] = start_w(i + 2)

            if not _NO_COMPUTE:
                with _scope(f"dot#b={b}"):
                    blk = jnp.dot(x8[...], w8[b, :, :],
                                  preferred_element_type=jnp.float32)
                    if b < N_DEV - 1:
                        send_buf[b, :, :] = blk.astype(jnp.bfloat16)
                    else:
                        out_ref[pl.ds(my * m_per, m_per), :] = blk * scale
            if b < N_DEV - 1 and not _NO_COMM:
                with _scope(f"send#b={b}"):
                    rdma = pltpu.make_async_remote_copy(
                        src_ref=send_buf.at[b],
                        dst_ref=recv_buf.at[b],
                        send_sem=send_sems.at[b],
                        recv_sem=recv_sems.at[b],
                        device_id=(j,),
                        device_id_type=pl.DeviceIdType.MESH,
                    )
                    rdma.start()
                    sends.append(rdma)

        for b in range(N_DEV - 1):
            s = (my - 1 - b) % N_DEV
            recv = pltpu.make_async_remote_copy(
                src_ref=send_buf.at[b],
                dst_ref=recv_buf.at[b],
                send_sem=send_sems.at[b],
                recv_sem=recv_sems.at[b],
                device_id=(s,),
                device_id_type=pl.DeviceIdType.MESH,
            )
            if not _NO_COMM:
                with _scope(f"wait_recv#b={b}"):
                    recv.wait_recv()
            with _scope(f"store_recv#b={b}"):
                out_ref[pl.ds(s * m_per, m_per), :] = (
                    recv_buf[b, :, :].astype(jnp.float32) * scale)

        if not _NO_COMM:
            with _scope("wait_send"):
                for rdma in sends:
                    rdma.wait_send()

    return pl.pallas_call(
        body,
        out_shape=jax.ShapeDtypeStruct((m_tot, n_per), jnp.float32),
        in_specs=[
            pl.BlockSpec(memory_space=pl.ANY),
            pl.BlockSpec(memory_space=pl.ANY),
            pl.BlockSpec(memory_space=pltpu.SMEM),
            pl.BlockSpec(memory_space=pltpu.SMEM),
        ],
        out_specs=pl.BlockSpec(memory_space=pltpu.VMEM),
        scratch_shapes=[
            pltpu.VMEM((2, mx, k), jnp.float32),
            pltpu.VMEM((2, kw, n_per), jnp.float32),
            pltpu.VMEM((m_per, k), jnp.float8_e4m3fn),
            pltpu.VMEM((N_DEV, k, n_per), jnp.float8_e5m2),
            pltpu.VMEM((N_DEV - 1, m_per, n_per), jnp.bfloat16),
            pltpu.VMEM((N_DEV - 1, m_per, n_per), jnp.bfloat16),
            pltpu.SemaphoreType.DMA((2,)),
            pltpu.SemaphoreType.DMA((2,)),
            pltpu.SemaphoreType.DMA((N_DEV - 1,)),
            pltpu.SemaphoreType.DMA((N_DEV - 1,)),
        ],
        compiler_params=pltpu.CompilerParams(collective_id=0),
    )(x, w_mat, scale_x, scale_w)


# device time: 46269 ns/iter; 5.3678x vs baseline; 5.3678x over previous
import contextlib
import os

import jax
import jax.numpy as jnp
from jax import lax
from jax.experimental import pallas as pl
from jax.experimental.pallas import tpu as pltpu

_PROFILE = os.environ.get("KERNEL_PROFILE", "0") == "1"
_NO_COMM = os.environ.get("KERNEL_NO_COMM", "0") == "1"
_NO_COMPUTE = os.environ.get("KERNEL_NO_COMPUTE", "0") == "1"
_NO_LOAD = os.environ.get("KERNEL_NO_LOAD", "0") == "1"


def _scope(name):
    return jax.named_scope(name) if _PROFILE else contextlib.nullcontext()


N_DEV = 4
XC = 4
WC = 2
_DELTA = (1, 3, 2, 0)


def kernel(x, w_mat, scale_x, scale_w):
    m_per, k = x.shape
    _, n = w_mat.shape
    n_per = n // N_DEV
    m_tot = m_per * N_DEV
    mx = m_per // XC
    kw = k // WC
    NW = N_DEV * WC

    def body(x_hbm, w_hbm, sx_ref, sw_ref, out_hbm,
             xstage, wstage, x8, w8, send_buf, recv_buf, outstage,
             xsems, wsems, send_sems, recv_sems, osems):
        my = lax.axis_index("i")

        with _scope("barrier"):
            barrier_sem = pltpu.get_barrier_semaphore()
            for h in range(1, N_DEV):
                pl.semaphore_signal(
                    barrier_sem, inc=1,
                    device_id=((my + h) % N_DEV,),
                    device_id_type=pl.DeviceIdType.MESH,
                )
            pl.semaphore_wait(barrier_sem, N_DEV - 1)

        scale = sx_ref[0] * sw_ref[0]

        def start_x(c):
            cp = pltpu.make_async_copy(
                x_hbm.at[pl.ds(c * mx, mx), :], xstage.at[c], xsems.at[c])
            cp.start()
            return cp

        def start_w(i):
            b, c = divmod(i, WC)
            j = (my + _DELTA[b]) % N_DEV
            cp = pltpu.make_async_copy(
                w_hbm.at[pl.ds(c * kw, kw), pl.ds(j * n_per, n_per)],
                wstage.at[i % 3], wsems.at[i % 3])
            cp.start()
            return cp

        xcp = {}
        wcp = {}
        if not _NO_LOAD:
            wcp[0] = start_w(0)
            wcp[1] = start_w(1)
            xcp[0] = start_x(0)
            xcp[1] = start_x(1)
            wcp[2] = start_w(2)

        def drain_x(c):
            if _NO_LOAD:
                return
            xcp[c].wait()
            x8[pl.ds(c * mx, mx), :] = xstage[c, :, :].astype(
                jnp.float8_e4m3fn)
            if c + 2 < XC:
                xcp[c + 2] = start_x(c + 2)

        def drain_w(i):
            if _NO_LOAD:
                return
            b, c = divmod(i, WC)
            wcp[i].wait()
            w8[b, pl.ds(c * kw, kw), :] = wstage[i % 3, :, :].astype(
                jnp.float8_e5m2)
            if i + 3 < NW:
                wcp[i + 3] = start_w(i + 3)

        sends = []

        def dot_send(c, b):
            j = (my + _DELTA[b]) % N_DEV
            if not _NO_COMPUTE:
                blk = jnp.dot(x8[pl.ds(c * mx, mx), :], w8[b, :, :],
                              preferred_element_type=jnp.float32)
                send_buf[b, pl.ds(c * mx, mx), :] = blk.astype(jnp.bfloat16)
            if not _NO_COMM:
                rdma = pltpu.make_async_remote_copy(
                    src_ref=send_buf.at[b, pl.ds(c * mx, mx), :],
                    dst_ref=recv_buf.at[b, pl.ds(c * mx, mx), :],
                    send_sem=send_sems.at[b, c],
                    recv_sem=recv_sems.at[b, c],
                    device_id=(j,),
                    device_id_type=pl.DeviceIdType.MESH,
                )
                rdma.start()
                sends.append(rdma)

        with _scope("pipe"):
            drain_w(0)
            drain_w(1)
            drain_x(0)
            dot_send(0, 0)
            drain_w(2)
            drain_w(3)
            dot_send(0, 1)
            drain_x(1)
            dot_send(1, 0)
            dot_send(1, 1)
            drain_x(2)
            dot_send(2, 0)
            dot_send(2, 1)
            drain_x(3)
            dot_send(3, 0)
            dot_send(3, 1)
            drain_w(4)
            drain_w(5)
            dot_send(0, 2)
            dot_send(1, 2)
            dot_send(2, 2)
            dot_send(3, 2)
            drain_w(6)
            drain_w(7)

        odma = [None, None]
        if not _NO_COMPUTE:
            with _scope("own_dot"):
                outstage[0, :, :] = jnp.dot(
                    x8[...], w8[N_DEV - 1, :, :],
                    preferred_element_type=jnp.float32) * scale
                odma[0] = pltpu.make_async_copy(
                    outstage.at[0],
                    out_hbm.at[pl.ds(my * m_per, m_per), :],
                    osems.at[0])
                odma[0].start()

        for b in range(N_DEV - 1):
            s = (my - _DELTA[b]) % N_DEV
            if not _NO_COMM:
                with _scope(f"wait_recv#b={b}"):
                    for c in range(XC):
                        recv = pltpu.make_async_remote_copy(
                            src_ref=send_buf.at[b, pl.ds(c * mx, mx), :],
                            dst_ref=recv_buf.at[b, pl.ds(c * mx, mx), :],
                            send_sem=send_sems.at[b, c],
                            recv_sem=recv_sems.at[b, c],
                            device_id=(s,),
                            device_id_type=pl.DeviceIdType.MESH,
                        )
                        recv.wait_recv()
            slot = (b + 1) % 2
            with _scope(f"store_recv#b={b}"):
                if odma[slot] is not None:
                    odma[slot].wait()
                outstage[slot, :, :] = (
                    recv_buf[b, :, :].astype(jnp.float32) * scale)
                odma[slot] = pltpu.make_async_copy(
                    outstage.at[slot],
                    out_hbm.at[pl.ds(s * m_per, m_per), :],
                    osems.at[slot])
                odma[slot].start()

        with _scope("tail_waits"):
            for d in odma:
                if d is not None:
                    d.wait()
            for rdma in sends:
                rdma.wait_send()

    return pl.pallas_call(
        body,
        out_shape=jax.ShapeDtypeStruct((m_tot, n_per), jnp.float32),
        in_specs=[
            pl.BlockSpec(memory_space=pl.ANY),
            pl.BlockSpec(memory_space=pl.ANY),
            pl.BlockSpec(memory_space=pltpu.SMEM),
            pl.BlockSpec(memory_space=pltpu.SMEM),
        ],
        out_specs=pl.BlockSpec(memory_space=pl.ANY),
        scratch_shapes=[
            pltpu.VMEM((XC, mx, k), jnp.float32),
            pltpu.VMEM((3, kw, n_per), jnp.float32),
            pltpu.VMEM((m_per, k), jnp.float8_e4m3fn),
            pltpu.VMEM((N_DEV, k, n_per), jnp.float8_e5m2),
            pltpu.VMEM((N_DEV - 1, m_per, n_per), jnp.bfloat16),
            pltpu.VMEM((N_DEV - 1, m_per, n_per), jnp.bfloat16),
            pltpu.VMEM((2, m_per, n_per), jnp.float32),
            pltpu.SemaphoreType.DMA((XC,)),
            pltpu.SemaphoreType.DMA((3,)),
            pltpu.SemaphoreType.DMA((N_DEV - 1, XC)),
            pltpu.SemaphoreType.DMA((N_DEV - 1, XC)),
            pltpu.SemaphoreType.DMA((2,)),
        ],
        compiler_params=pltpu.CompilerParams(
            collective_id=0,
            vmem_limit_bytes=64 * 1024 * 1024,
        ),
    )(x, w_mat, scale_x, scale_w)
